# baseline (speedup 1.0000x reference)
import functools

import jax
import jax.numpy as jnp
import ml_dtypes
import numpy as np
from jax import lax
from jax.experimental import pallas as pl
from jax.experimental.pallas import tpu as pltpu
from jax.experimental.pallas import tpu_sc as plsc

N_PROT = 10000
N_LIG = 10000
E = 160000
RADIUS_EMB_DIM = 64
FOLD_DIM = 256
PROTEIN_RADIUS = 8.0

NC = 2
NS = 16
NW = NC * NS
CHUNK = 128
ROUND = 8
RB = ROUND * CHUNK

NPART = 5
PART_E = E // NPART
PART_PAD = NW * RB

IN_DIM = RADIUS_EMB_DIM * 5
SPACING = PROTEIN_RADIUS / (RADIUS_EMB_DIM - 1)
COEFF = -0.5 / SPACING**2

TC_BLOCK = 6400


def _sc_body(lt_hbm, pt_hbm, src_hbm, dst_hbm, d2_hbm,
             sidx, didx, lrows, prows, d2buf, sem_l, sem_p):
    wid = lax.axis_index("s") * NC + lax.axis_index("c")
    ii = lax.iota(jnp.int32, 16)
    c0 = jnp.zeros((16,), jnp.int32)
    base = wid * ROUND

    pltpu.sync_copy(src_hbm.at[pl.ds(base, ROUND)], sidx)
    pltpu.sync_copy(dst_hbm.at[pl.ds(base, ROUND)], didx)
    copies = []
    for j in range(ROUND):
        copies.append(pltpu.async_copy(lt_hbm.at[sidx.at[j]], lrows.at[j], sem_l))
        copies.append(pltpu.async_copy(pt_hbm.at[didx.at[j]], prows.at[j], sem_p))
    for c in copies:
        c.wait()

    def group_body(g, _):
        cj = jnp.broadcast_to(g // 8, (16,))
        rowi = ii + (g % 8) * 16
        lxyz = [plsc.load_gather(lrows, [cj, rowi, c0 + c]) for c in range(3)]
        for k in range(5):
            pxyz = [plsc.load_gather(prows, [cj, rowi, c0 + (3 * k + c)])
                    for c in range(3)]
            dx = lxyz[0] - pxyz[0]
            dy = lxyz[1] - pxyz[1]
            dz = lxyz[2] - pxyz[2]
            d2buf[k, pl.ds(g * 16, 16)] = dx * dx + dy * dy + dz * dz
        return _

    lax.fori_loop(0, RB // 16, group_body, None)
    for k in range(5):
        pltpu.sync_copy(d2buf.at[k], d2_hbm.at[k, pl.ds(base * CHUNK, RB)])


def _sc_dist2_part(ltab, ptab, srcp, dstp):
    f = pl.kernel(
        _sc_body,
        out_type=jax.ShapeDtypeStruct((5, PART_PAD), jnp.float32),
        mesh=plsc.VectorSubcoreMesh(
            core_axis_name="c", subcore_axis_name="s",
            num_cores=NC, num_subcores=NS),
        scratch_types=[
            pltpu.VMEM((ROUND, CHUNK), jnp.int32),
            pltpu.VMEM((ROUND, CHUNK), jnp.int32),
            pltpu.VMEM((ROUND, CHUNK, 16), jnp.float32),
            pltpu.VMEM((ROUND, CHUNK, 16), jnp.float32),
            pltpu.VMEM((5, RB), jnp.float32),
            pltpu.SemaphoreType.DMA,
            pltpu.SemaphoreType.DMA,
        ],
        compiler_params=pltpu.CompilerParams(
            use_tc_tiling_on_sc=False, needs_layout_passes=False),
    )
    return f(ltab, ptab, srcp, dstp)


def _tc_mlp_body(d2_ref, zsel_ref, w1_ref, b1_ref, w2_ref, b2_ref, *rest):
    out_ref = rest[-1]
    d2 = d2_ref[...]
    d2c = jnp.maximum(d2, 1e-24)
    r = lax.rsqrt(d2c)
    r = r * (1.5 - 0.5 * d2c * r * r)
    r = r * (1.5 - 0.5 * d2c * r * r)
    d = d2 * r
    d_hi = d.astype(jnp.bfloat16).astype(jnp.float32)
    d_lo = d - d_hi
    ones = jnp.ones((2, d.shape[1]), jnp.float32)
    aug = jnp.concatenate([d_hi, d_lo, ones], axis=0)
    z = jnp.dot(jnp.transpose(aug), zsel_ref[...],
                preferred_element_type=jnp.float32)
    att = jnp.exp(COEFF * z * z)
    h = jnp.maximum(
        jnp.dot(att, w1_ref[...], preferred_element_type=jnp.float32)
        + b1_ref[...], 0.0)
    out_ref[...] = (
        jnp.dot(h, w2_ref[...], preferred_element_type=jnp.float32)
        + b2_ref[...])


def _tc_mlp_part(part, d2p, zsel, W1, b1, W2, b2, prev=None):
    nblk = PART_E // TC_BLOCK
    in_specs = [
        pl.BlockSpec((5, TC_BLOCK), lambda i: (0, i)),
        pl.BlockSpec((12, IN_DIM), lambda i: (0, 0)),
        pl.BlockSpec((IN_DIM, FOLD_DIM), lambda i: (0, 0)),
        pl.BlockSpec((1, FOLD_DIM), lambda i: (0, 0)),
        pl.BlockSpec((FOLD_DIM, FOLD_DIM), lambda i: (0, 0)),
        pl.BlockSpec((1, FOLD_DIM), lambda i: (0, 0)),
    ]
    args = [d2p, zsel, W1, b1, W2, b2]
    aliases = {}
    if prev is not None:
        in_specs.append(pl.BlockSpec(memory_space=pl.ANY))
        args.append(prev)
        aliases = {6: 0}
    return pl.pallas_call(
        _tc_mlp_body,
        grid=(nblk,),
        in_specs=in_specs,
        out_specs=pl.BlockSpec((TC_BLOCK, FOLD_DIM),
                               lambda i: (i + part * nblk, 0)),
        out_shape=jax.ShapeDtypeStruct((E, FOLD_DIM), jnp.float32),
        input_output_aliases=aliases,
    )(*args)


def _z_selector():
    offs = np.linspace(0.0, PROTEIN_RADIUS, RADIUS_EMB_DIM,
                       dtype=np.float32)
    offs320 = np.tile(offs, 5)
    hi = offs320.astype(ml_dtypes.bfloat16).astype(np.float32)
    lo = offs320 - hi
    s = np.zeros((12, IN_DIM), dtype=np.float32)
    for k in range(5):
        s[k, 64 * k:64 * (k + 1)] = 1.0
        s[5 + k, 64 * k:64 * (k + 1)] = 1.0
    s[10] = -hi
    s[11] = -lo
    return jnp.asarray(s)


def kernel(ligand_pos, protein_pos, protein_pos_Cb, protein_pos_C,
           protein_pos_O, protein_pos_N, edge_index, W1, b1, W2, b2):
    zpad = jnp.zeros((N_PROT, 1), jnp.float32)
    ptab = jnp.concatenate(
        [protein_pos, protein_pos_Cb, protein_pos_C, protein_pos_O,
         protein_pos_N, zpad], axis=1)
    ltab = jnp.concatenate(
        [ligand_pos, jnp.zeros((N_LIG, 13), jnp.float32)], axis=1)

    ipad = jnp.zeros((NPART, PART_PAD - PART_E), jnp.int32)
    src5 = jnp.concatenate([edge_index[0].reshape(NPART, PART_E), ipad],
                           axis=1)
    dst5 = jnp.concatenate([edge_index[1].reshape(NPART, PART_E), ipad],
                           axis=1)

    d2parts = [
        _sc_dist2_part(ltab, ptab,
                       src5[p].reshape(NW * ROUND, CHUNK),
                       dst5[p].reshape(NW * ROUND, CHUNK))
        for p in range(NPART)
    ]

    zsel = _z_selector()
    b1r = b1.reshape(1, FOLD_DIM)
    b2r = b2.reshape(1, FOLD_DIM)
    out = _tc_mlp_part(0, d2parts[0], zsel, W1, b1r, W2, b2r)
    for p in range(1, NPART):
        out = _tc_mlp_part(p, d2parts[p], zsel, W1, b1r, W2, b2r, prev=out)
    return (edge_index, out)


# speedup vs baseline: 1.2674x; 1.2674x over previous
import functools

import jax
import jax.numpy as jnp
import ml_dtypes
import numpy as np
from jax import lax
from jax.experimental import pallas as pl
from jax.experimental.pallas import tpu as pltpu
from jax.experimental.pallas import tpu_sc as plsc

N_PROT = 10000
N_LIG = 10000
E = 160000
RADIUS_EMB_DIM = 64
FOLD_DIM = 256
PROTEIN_RADIUS = 8.0

NC = 2
NS = 16
NW = NC * NS
CHUNK = 128
CPW = 40
ROUND = 8
NROUND = CPW // ROUND
RB = ROUND * CHUNK
EPAD = NW * CPW * CHUNK

IN_DIM = RADIUS_EMB_DIM * 5
SPACING = PROTEIN_RADIUS / (RADIUS_EMB_DIM - 1)
COEFF = -0.5 / SPACING**2

TC_BLOCK = 6400


def _sc_body(lt_hbm, pt_hbm, src_hbm, dst_hbm, d2_hbm,
             sidx, didx, lrows, prows, d2buf, sem_l0, sem_l1, sem_p0, sem_p1):
    wid = lax.axis_index("s") * NC + lax.axis_index("c")
    ii = lax.iota(jnp.int32, 16)
    c0 = jnp.zeros((16,), jnp.int32)
    sem_l = [sem_l0, sem_l1]
    sem_p = [sem_p0, sem_p1]

    def fire(r, buf):
        base = wid * CPW + r * ROUND
        pltpu.sync_copy(src_hbm.at[pl.ds(base * CHUNK, RB)], sidx.at[buf])
        pltpu.sync_copy(dst_hbm.at[pl.ds(base * CHUNK, RB)], didx.at[buf])
        return [
            pltpu.async_copy(lt_hbm.at[sidx.at[buf]], lrows.at[buf], sem_l[buf]),
            pltpu.async_copy(pt_hbm.at[didx.at[buf]], prows.at[buf], sem_p[buf]),
        ]

    def make_group_body(buf):
        def group_body(g, _):
            rowi = ii + g * 16
            lxyz = [plsc.load_gather(lrows, [c0 + buf, rowi, c0 + c])
                    for c in range(3)]
            for k in range(5):
                pxyz = [plsc.load_gather(prows,
                                         [c0 + buf, rowi, c0 + (3 * k + c)])
                        for c in range(3)]
                dx = lxyz[0] - pxyz[0]
                dy = lxyz[1] - pxyz[1]
                dz = lxyz[2] - pxyz[2]
                d2buf[k, pl.ds(g * 16, 16)] = dx * dx + dy * dy + dz * dz
            return _
        return group_body

    pend = fire(0, 0)
    for r in range(NROUND):
        if r + 1 < NROUND:
            nxt = fire(r + 1, (r + 1) % 2)
        else:
            nxt = []
        for c in pend:
            c.wait()
        lax.fori_loop(0, RB // 16, make_group_body(r % 2), None)
        base = wid * CPW + r * ROUND
        for k in range(5):
            pltpu.sync_copy(d2buf.at[k], d2_hbm.at[k, pl.ds(base * CHUNK, RB)])
        pend = nxt


def _sc_dist2(ltab, ptab, src3, dst3):
    f = pl.kernel(
        _sc_body,
        out_type=jax.ShapeDtypeStruct((5, EPAD), jnp.float32),
        mesh=plsc.VectorSubcoreMesh(
            core_axis_name="c", subcore_axis_name="s",
            num_cores=NC, num_subcores=NS),
        scratch_types=[
            pltpu.VMEM((2, RB), jnp.int32),
            pltpu.VMEM((2, RB), jnp.int32),
            pltpu.VMEM((2, RB, 16), jnp.float32),
            pltpu.VMEM((2, RB, 16), jnp.float32),
            pltpu.VMEM((5, RB), jnp.float32),
            pltpu.SemaphoreType.DMA,
            pltpu.SemaphoreType.DMA,
            pltpu.SemaphoreType.DMA,
            pltpu.SemaphoreType.DMA,
        ],
        compiler_params=pltpu.CompilerParams(
            use_tc_tiling_on_sc=False, needs_layout_passes=False),
    )
    return f(ltab, ptab, src3, dst3)


def _tc_mlp_body(d2_ref, zsel_ref, w1_ref, b1_ref,
                 w2_ref, b2_ref, out_ref):
    d2 = d2_ref[...]
    d2c = jnp.maximum(d2, 1e-24)
    r = lax.rsqrt(d2c)
    r = r * (1.5 - 0.5 * d2c * r * r)
    r = r * (1.5 - 0.5 * d2c * r * r)
    d = d2 * r
    d_hi = d.astype(jnp.bfloat16).astype(jnp.float32)
    d_lo = d - d_hi
    ones = jnp.ones((2, d.shape[1]), jnp.float32)
    aug = jnp.concatenate([d_hi, d_lo, ones], axis=0)
    z = jnp.dot(jnp.transpose(aug), zsel_ref[...],
                preferred_element_type=jnp.float32)
    att = jnp.exp(COEFF * z * z)
    h = jnp.maximum(
        jnp.dot(att, w1_ref[...], preferred_element_type=jnp.float32)
        + b1_ref[...], 0.0)
    out_ref[...] = (
        jnp.dot(h, w2_ref[...], preferred_element_type=jnp.float32)
        + b2_ref[...])


def _tc_mlp(d2, zsel, W1, b1, W2, b2):
    grid = (E // TC_BLOCK,)
    return pl.pallas_call(
        _tc_mlp_body,
        grid=grid,
        in_specs=[
            pl.BlockSpec((5, TC_BLOCK), lambda i: (0, i)),
            pl.BlockSpec((12, IN_DIM), lambda i: (0, 0)),
            pl.BlockSpec((IN_DIM, FOLD_DIM), lambda i: (0, 0)),
            pl.BlockSpec((1, FOLD_DIM), lambda i: (0, 0)),
            pl.BlockSpec((FOLD_DIM, FOLD_DIM), lambda i: (0, 0)),
            pl.BlockSpec((1, FOLD_DIM), lambda i: (0, 0)),
        ],
        out_specs=pl.BlockSpec((TC_BLOCK, FOLD_DIM), lambda i: (i, 0)),
        out_shape=jax.ShapeDtypeStruct((E, FOLD_DIM), jnp.float32),
    )(d2, zsel, W1, b1, W2, b2)


def _z_selector():
    offs = np.linspace(0.0, PROTEIN_RADIUS, RADIUS_EMB_DIM,
                       dtype=np.float32)
    offs320 = np.tile(offs, 5)
    hi = offs320.astype(ml_dtypes.bfloat16).astype(np.float32)
    lo = offs320 - hi
    s = np.zeros((12, IN_DIM), dtype=np.float32)
    for k in range(5):
        s[k, 64 * k:64 * (k + 1)] = 1.0
        s[5 + k, 64 * k:64 * (k + 1)] = 1.0
    s[10] = -hi
    s[11] = -lo
    return jnp.asarray(s)


def kernel(ligand_pos, protein_pos, protein_pos_Cb, protein_pos_C,
           protein_pos_O, protein_pos_N, edge_index, W1, b1, W2, b2):
    zpad = jnp.zeros((N_PROT, 1), jnp.float32)
    ptab = jnp.concatenate(
        [protein_pos, protein_pos_Cb, protein_pos_C, protein_pos_O,
         protein_pos_N, zpad], axis=1)
    ltab = jnp.concatenate(
        [ligand_pos, jnp.zeros((N_LIG, 13), jnp.float32)], axis=1)

    ipad = jnp.zeros((EPAD - E,), jnp.int32)
    src3 = jnp.concatenate([edge_index[0], ipad])
    dst3 = jnp.concatenate([edge_index[1], ipad])

    d2 = _sc_dist2(ltab, ptab, src3, dst3)

    out = _tc_mlp(d2, _z_selector(), W1, b1.reshape(1, FOLD_DIM),
                  W2, b2.reshape(1, FOLD_DIM))
    return (edge_index, out)


# speedup vs baseline: 1.3381x; 1.0558x over previous
import functools

import jax
import jax.numpy as jnp
import ml_dtypes
import numpy as np
from jax import lax
from jax.experimental import pallas as pl
from jax.experimental.pallas import tpu as pltpu
from jax.experimental.pallas import tpu_sc as plsc

N_PROT = 10000
N_LIG = 10000
E = 160000
RADIUS_EMB_DIM = 64
FOLD_DIM = 256
PROTEIN_RADIUS = 8.0

NC = 2
NS = 16
NW = NC * NS
CHUNK = 128
CPW = 40
ROUND = 8
NROUND = CPW // ROUND
RB = ROUND * CHUNK
EPAD = NW * CPW * CHUNK

IN_DIM = RADIUS_EMB_DIM * 5
SPACING = PROTEIN_RADIUS / (RADIUS_EMB_DIM - 1)
COEFF = -0.5 / SPACING**2

TC_BLOCK = 6400


def _sc_body(lt_hbm, pt_hbm, ei_hbm, d2_hbm,
             sidx, didx, lrows, prows, d2buf, sem_l0, sem_l1, sem_p0, sem_p1):
    wid = lax.axis_index("s") * NC + lax.axis_index("c")
    ii = lax.iota(jnp.int32, 16)
    c0 = jnp.zeros((16,), jnp.int32)
    sem_l = [sem_l0, sem_l1]
    sem_p = [sem_p0, sem_p1]

    def rbase(r):
        return jnp.minimum(wid * (CPW * CHUNK) + r * RB, E - RB)

    def fire(r, buf):
        base = rbase(r)
        pltpu.sync_copy(ei_hbm.at[0, pl.ds(base, RB)], sidx.at[buf])
        pltpu.sync_copy(ei_hbm.at[1, pl.ds(base, RB)], didx.at[buf])
        return [
            pltpu.async_copy(lt_hbm.at[sidx.at[buf]], lrows.at[buf], sem_l[buf]),
            pltpu.async_copy(pt_hbm.at[didx.at[buf]], prows.at[buf], sem_p[buf]),
        ]

    def make_group_body(buf):
        def group_body(g, _):
            rowi = ii + g * 16
            lxyz = [plsc.load_gather(lrows, [c0 + buf, rowi, c0 + c])
                    for c in range(3)]
            for k in range(5):
                pxyz = [plsc.load_gather(prows,
                                         [c0 + buf, rowi, c0 + (3 * k + c)])
                        for c in range(3)]
                dx = lxyz[0] - pxyz[0]
                dy = lxyz[1] - pxyz[1]
                dz = lxyz[2] - pxyz[2]
                d2buf[k, pl.ds(g * 16, 16)] = dx * dx + dy * dy + dz * dz
            return _
        return group_body

    pend = fire(0, 0)
    for r in range(NROUND):
        if r + 1 < NROUND:
            nxt = fire(r + 1, (r + 1) % 2)
        else:
            nxt = []
        for c in pend:
            c.wait()
        lax.fori_loop(0, RB // 16, make_group_body(r % 2), None)
        base = rbase(r)
        for k in range(5):
            pltpu.sync_copy(d2buf.at[k], d2_hbm.at[k, pl.ds(base, RB)])
        pend = nxt


def _sc_dist2(ltab, ptab, edge_index):
    f = pl.kernel(
        _sc_body,
        out_type=jax.ShapeDtypeStruct((5, E), jnp.float32),
        mesh=plsc.VectorSubcoreMesh(
            core_axis_name="c", subcore_axis_name="s",
            num_cores=NC, num_subcores=NS),
        scratch_types=[
            pltpu.VMEM((2, RB), jnp.int32),
            pltpu.VMEM((2, RB), jnp.int32),
            pltpu.VMEM((2, RB, 16), jnp.float32),
            pltpu.VMEM((2, RB, 16), jnp.float32),
            pltpu.VMEM((5, RB), jnp.float32),
            pltpu.SemaphoreType.DMA,
            pltpu.SemaphoreType.DMA,
            pltpu.SemaphoreType.DMA,
            pltpu.SemaphoreType.DMA,
        ],
        compiler_params=pltpu.CompilerParams(
            use_tc_tiling_on_sc=False, needs_layout_passes=False),
    )
    return f(ltab, ptab, edge_index)


def _tc_mlp_body(d2_ref, zsel_ref, w1_ref, b1_ref,
                 w2_ref, b2_ref, out_ref):
    d2 = d2_ref[...]
    d2c = jnp.maximum(d2, 1e-24)
    r = lax.rsqrt(d2c)
    r = r * (1.5 - 0.5 * d2c * r * r)
    r = r * (1.5 - 0.5 * d2c * r * r)
    d = d2 * r
    d_hi = d.astype(jnp.bfloat16).astype(jnp.float32)
    d_lo = d - d_hi
    ones = jnp.ones((2, d.shape[1]), jnp.float32)
    aug = jnp.concatenate([d_hi, d_lo, ones], axis=0)
    z = jnp.dot(jnp.transpose(aug), zsel_ref[...],
                preferred_element_type=jnp.float32)
    att = jnp.exp(COEFF * z * z)
    h = jnp.maximum(
        jnp.dot(att, w1_ref[...], preferred_element_type=jnp.float32)
        + b1_ref[...], 0.0)
    out_ref[...] = (
        jnp.dot(h, w2_ref[...], preferred_element_type=jnp.float32)
        + b2_ref[...])


def _tc_mlp(d2, zsel, W1, b1, W2, b2):
    grid = (E // TC_BLOCK,)
    return pl.pallas_call(
        _tc_mlp_body,
        grid=grid,
        in_specs=[
            pl.BlockSpec((5, TC_BLOCK), lambda i: (0, i)),
            pl.BlockSpec((12, IN_DIM), lambda i: (0, 0)),
            pl.BlockSpec((IN_DIM, FOLD_DIM), lambda i: (0, 0)),
            pl.BlockSpec((1, FOLD_DIM), lambda i: (0, 0)),
            pl.BlockSpec((FOLD_DIM, FOLD_DIM), lambda i: (0, 0)),
            pl.BlockSpec((1, FOLD_DIM), lambda i: (0, 0)),
        ],
        out_specs=pl.BlockSpec((TC_BLOCK, FOLD_DIM), lambda i: (i, 0)),
        out_shape=jax.ShapeDtypeStruct((E, FOLD_DIM), jnp.float32),
    )(d2, zsel, W1, b1, W2, b2)


def _z_selector():
    offs = np.linspace(0.0, PROTEIN_RADIUS, RADIUS_EMB_DIM,
                       dtype=np.float32)
    offs320 = np.tile(offs, 5)
    hi = offs320.astype(ml_dtypes.bfloat16).astype(np.float32)
    lo = offs320 - hi
    s = np.zeros((12, IN_DIM), dtype=np.float32)
    for k in range(5):
        s[k, 64 * k:64 * (k + 1)] = 1.0
        s[5 + k, 64 * k:64 * (k + 1)] = 1.0
    s[10] = -hi
    s[11] = -lo
    return jnp.asarray(s)


def kernel(ligand_pos, protein_pos, protein_pos_Cb, protein_pos_C,
           protein_pos_O, protein_pos_N, edge_index, W1, b1, W2, b2):
    zpad = jnp.zeros((N_PROT, 1), jnp.float32)
    ptab = jnp.concatenate(
        [protein_pos, protein_pos_Cb, protein_pos_C, protein_pos_O,
         protein_pos_N, zpad], axis=1)
    ltab = jnp.concatenate(
        [ligand_pos, jnp.zeros((N_LIG, 13), jnp.float32)], axis=1)

    d2 = _sc_dist2(ltab, ptab, edge_index)

    out = _tc_mlp(d2, _z_selector(), W1, b1.reshape(1, FOLD_DIM),
                  W2, b2.reshape(1, FOLD_DIM))
    return (edge_index, out)


# speedup vs baseline: 1.3473x; 1.0069x over previous
import functools

import jax
import jax.numpy as jnp
import ml_dtypes
import numpy as np
from jax import lax
from jax.experimental import pallas as pl
from jax.experimental.pallas import tpu as pltpu
from jax.experimental.pallas import tpu_sc as plsc

N_PROT = 10000
N_LIG = 10000
E = 160000
RADIUS_EMB_DIM = 64
FOLD_DIM = 256
PROTEIN_RADIUS = 8.0

NC = 2
NS = 16
NW = NC * NS
CHUNK = 128
CPW = 40
ROUND = 8
NROUND = CPW // ROUND
RB = ROUND * CHUNK
EPAD = NW * CPW * CHUNK

IN_DIM = RADIUS_EMB_DIM * 5
SPACING = PROTEIN_RADIUS / (RADIUS_EMB_DIM - 1)
COEFF = -0.5 / SPACING**2

TC_BLOCK = 6400


def _sc_body(lt_hbm, pt_hbm, ei_hbm, d2_hbm,
             sidx, didx, lrows, prows, d2buf,
             sem_l0, sem_l1, sem_p0, sem_p1, sem_w0, sem_w1):
    wid = lax.axis_index("s") * NC + lax.axis_index("c")
    ii = lax.iota(jnp.int32, 16)
    c0 = jnp.zeros((16,), jnp.int32)
    sem_l = [sem_l0, sem_l1]
    sem_p = [sem_p0, sem_p1]
    sem_w = [sem_w0, sem_w1]

    def rbase(r):
        return jnp.minimum(wid * (CPW * CHUNK) + r * RB, E - RB)

    def fire(r, buf):
        base = rbase(r)
        pltpu.sync_copy(ei_hbm.at[0, pl.ds(base, RB)], sidx.at[buf])
        pltpu.sync_copy(ei_hbm.at[1, pl.ds(base, RB)], didx.at[buf])
        return [
            pltpu.async_copy(lt_hbm.at[sidx.at[buf]], lrows.at[buf], sem_l[buf]),
            pltpu.async_copy(pt_hbm.at[didx.at[buf]], prows.at[buf], sem_p[buf]),
        ]

    def make_group_body(buf):
        def group_body(g, _):
            rowi = ii + g * 16
            lxyz = [plsc.load_gather(lrows, [c0 + buf, rowi, c0 + c])
                    for c in range(3)]
            for k in range(5):
                pxyz = [plsc.load_gather(prows,
                                         [c0 + buf, rowi, c0 + (3 * k + c)])
                        for c in range(3)]
                dx = lxyz[0] - pxyz[0]
                dy = lxyz[1] - pxyz[1]
                dz = lxyz[2] - pxyz[2]
                d2buf[buf, k, pl.ds(g * 16, 16)] = dx * dx + dy * dy + dz * dz
            return _
        return group_body

    pend = fire(0, 0)
    wslot = [[], []]
    for r in range(NROUND):
        if r + 1 < NROUND:
            nxt = fire(r + 1, (r + 1) % 2)
        else:
            nxt = []
        for c in pend:
            c.wait()
        for c in wslot[r % 2]:
            c.wait()
        lax.fori_loop(0, RB // 16, make_group_body(r % 2), None)
        base = rbase(r)
        wslot[r % 2] = [
            pltpu.async_copy(d2buf.at[r % 2, k],
                             d2_hbm.at[k, pl.ds(base, RB)], sem_w[r % 2])
            for k in range(5)
        ]
        pend = nxt
    for c in wslot[0] + wslot[1]:
        c.wait()


def _sc_dist2(ltab, ptab, edge_index):
    f = pl.kernel(
        _sc_body,
        out_type=jax.ShapeDtypeStruct((5, E), jnp.float32),
        mesh=plsc.VectorSubcoreMesh(
            core_axis_name="c", subcore_axis_name="s",
            num_cores=NC, num_subcores=NS),
        scratch_types=[
            pltpu.VMEM((2, RB), jnp.int32),
            pltpu.VMEM((2, RB), jnp.int32),
            pltpu.VMEM((2, RB, 16), jnp.float32),
            pltpu.VMEM((2, RB, 16), jnp.float32),
            pltpu.VMEM((2, 5, RB), jnp.float32),
            pltpu.SemaphoreType.DMA,
            pltpu.SemaphoreType.DMA,
            pltpu.SemaphoreType.DMA,
            pltpu.SemaphoreType.DMA,
            pltpu.SemaphoreType.DMA,
            pltpu.SemaphoreType.DMA,
        ],
        compiler_params=pltpu.CompilerParams(
            use_tc_tiling_on_sc=False, needs_layout_passes=False),
    )
    return f(ltab, ptab, edge_index)


def _tc_mlp_body(d2_ref, zsel_ref, w1_ref, b1_ref,
                 w2_ref, b2_ref, out_ref):
    d2 = d2_ref[...]
    d2c = jnp.maximum(d2, 1e-24)
    r = lax.rsqrt(d2c)
    r = r * (1.5 - 0.5 * d2c * r * r)
    r = r * (1.5 - 0.5 * d2c * r * r)
    d = d2 * r
    d_hi = d.astype(jnp.bfloat16).astype(jnp.float32)
    d_lo = d - d_hi
    ones = jnp.ones((2, d.shape[1]), jnp.float32)
    aug = jnp.concatenate([d_hi, d_lo, ones], axis=0)
    z = jnp.dot(jnp.transpose(aug), zsel_ref[...],
                preferred_element_type=jnp.float32)
    att = jnp.exp(COEFF * z * z)
    h = jnp.maximum(
        jnp.dot(att, w1_ref[...], preferred_element_type=jnp.float32)
        + b1_ref[...], 0.0)
    out_ref[...] = (
        jnp.dot(h, w2_ref[...], preferred_element_type=jnp.float32)
        + b2_ref[...])


def _tc_mlp(d2, zsel, W1, b1, W2, b2):
    grid = (E // TC_BLOCK,)
    return pl.pallas_call(
        _tc_mlp_body,
        grid=grid,
        in_specs=[
            pl.BlockSpec((5, TC_BLOCK), lambda i: (0, i)),
            pl.BlockSpec((12, IN_DIM), lambda i: (0, 0)),
            pl.BlockSpec((IN_DIM, FOLD_DIM), lambda i: (0, 0)),
            pl.BlockSpec((1, FOLD_DIM), lambda i: (0, 0)),
            pl.BlockSpec((FOLD_DIM, FOLD_DIM), lambda i: (0, 0)),
            pl.BlockSpec((1, FOLD_DIM), lambda i: (0, 0)),
        ],
        out_specs=pl.BlockSpec((TC_BLOCK, FOLD_DIM), lambda i: (i, 0)),
        out_shape=jax.ShapeDtypeStruct((E, FOLD_DIM), jnp.float32),
    )(d2, zsel, W1, b1, W2, b2)


def _z_selector():
    offs = np.linspace(0.0, PROTEIN_RADIUS, RADIUS_EMB_DIM,
                       dtype=np.float32)
    offs320 = np.tile(offs, 5)
    hi = offs320.astype(ml_dtypes.bfloat16).astype(np.float32)
    lo = offs320 - hi
    s = np.zeros((12, IN_DIM), dtype=np.float32)
    for k in range(5):
        s[k, 64 * k:64 * (k + 1)] = 1.0
        s[5 + k, 64 * k:64 * (k + 1)] = 1.0
    s[10] = -hi
    s[11] = -lo
    return jnp.asarray(s)


def kernel(ligand_pos, protein_pos, protein_pos_Cb, protein_pos_C,
           protein_pos_O, protein_pos_N, edge_index, W1, b1, W2, b2):
    zpad = jnp.zeros((N_PROT, 1), jnp.float32)
    ptab = jnp.concatenate(
        [protein_pos, protein_pos_Cb, protein_pos_C, protein_pos_O,
         protein_pos_N, zpad], axis=1)
    ltab = jnp.concatenate(
        [ligand_pos, jnp.zeros((N_LIG, 13), jnp.float32)], axis=1)

    d2 = _sc_dist2(ltab, ptab, edge_index)

    out = _tc_mlp(d2, _z_selector(), W1, b1.reshape(1, FOLD_DIM),
                  W2, b2.reshape(1, FOLD_DIM))
    return (edge_index, out)
